# baseline (speedup 1.0000x reference)
import math

import jax
import jax.numpy as jnp
from jax import lax
from jax.experimental import pallas as pl
from jax.experimental.pallas import tpu as pltpu

_C = 100000
_B = 1024
_SMOOTHING = 0.1
_CONF = 1.0 - _SMOOTHING
_S = _SMOOTHING / (_C - 1)
_W = 1024
_NBT = _C // _W
_TAIL = _C - _NBT * _W
_NBUF = 8


def _copy(pred_ref, bufs, sems, blk, slot):
    pltpu.make_async_copy(
        pred_ref.at[:, pl.ds(blk * _W, _W)], bufs.at[slot], sems.at[slot]
    ).start()


def _loss_kernel(tgt_ref, pred_ref, out_ref, bufs, tail_buf,
                 m_ref, se_ref, rs_ref, g_ref, sems, tail_sem):
    m_ref[...] = jnp.full_like(m_ref, float("-inf"))
    se_ref[...] = jnp.zeros_like(se_ref)
    rs_ref[...] = jnp.zeros_like(rs_ref)
    g_ref[...] = jnp.zeros_like(g_ref)

    for b in range(_NBUF):
        _copy(pred_ref, bufs, sems, b, b)
    pltpu.make_async_copy(
        pred_ref.at[:, pl.ds(_NBT * _W, _TAIL)], tail_buf, tail_sem
    ).start()

    tgt = tgt_ref[...]

    def _update(x, col0):
        col = jax.lax.broadcasted_iota(jnp.int32, x.shape, 1)
        m = m_ref[...]
        bm = jnp.max(x, axis=1, keepdims=True)
        nm = jnp.maximum(m, bm)
        se_ref[...] = se_ref[...] * jnp.exp(m - nm) + jnp.sum(
            jnp.exp(x - nm), axis=1, keepdims=True
        )
        m_ref[...] = nm
        rs_ref[...] += jnp.sum(x, axis=1, keepdims=True)
        g_ref[...] += jnp.sum(
            jnp.where(col == tgt - col0, x, 0.0), axis=1, keepdims=True
        )

    def _step(blk, _):
        slot = lax.rem(blk, _NBUF)
        pltpu.make_async_copy(
            pred_ref.at[:, pl.ds(blk * _W, _W)], bufs.at[slot], sems.at[slot]
        ).wait()
        _update(bufs[slot], blk * _W)
        nxt = blk + _NBUF

        @pl.when(nxt < _NBT)
        def _start_next():
            _copy(pred_ref, bufs, sems, nxt, slot)

        return _

    lax.fori_loop(0, _NBT, _step, 0)

    pltpu.make_async_copy(
        pred_ref.at[:, pl.ds(_NBT * _W, _TAIL)], tail_buf, tail_sem
    ).wait()
    _update(tail_buf[...], _NBT * _W)

    z = m_ref[...] + jnp.log(se_ref[...])
    zsum = jnp.sum(z)
    kc = _SMOOTHING * math.log(_S) + _CONF * math.log(_CONF)
    total = (
        _B * kc
        - _S * (jnp.sum(rs_ref[...]) - _C * zsum)
        - (_CONF - _S) * (jnp.sum(g_ref[...]) - zsum)
    )
    out_ref[0, 0] = total / (_B * _C)


def kernel(pred, target):
    tgt = target.astype(jnp.int32).reshape(_B, 1)
    out = pl.pallas_call(
        _loss_kernel,
        in_specs=[
            pl.BlockSpec((_B, 1), lambda: (0, 0)),
            pl.BlockSpec(memory_space=pl.ANY),
        ],
        out_specs=pl.BlockSpec(
            (1, 1), lambda: (0, 0), memory_space=pltpu.SMEM
        ),
        out_shape=jax.ShapeDtypeStruct((1, 1), jnp.float32),
        scratch_shapes=[
            pltpu.VMEM((_NBUF, _B, _W), jnp.float32),
            pltpu.VMEM((_B, _TAIL), jnp.float32),
            pltpu.VMEM((_B, 1), jnp.float32),
            pltpu.VMEM((_B, 1), jnp.float32),
            pltpu.VMEM((_B, 1), jnp.float32),
            pltpu.VMEM((_B, 1), jnp.float32),
            pltpu.SemaphoreType.DMA((_NBUF,)),
            pltpu.SemaphoreType.DMA,
        ],
    )(tgt, pred)
    return out[0, 0]


# speedup vs baseline: 1.0002x; 1.0002x over previous
import math

import jax
import jax.numpy as jnp
from jax import lax
from jax.experimental import pallas as pl
from jax.experimental.pallas import tpu as pltpu

_C = 100000
_B = 1024
_SMOOTHING = 0.1
_CONF = 1.0 - _SMOOTHING
_S = _SMOOTHING / (_C - 1)
_W = 1024
_NBT = _C // _W
_TAIL = _C - _NBT * _W
_NBUF = 12


def _copy(pred_ref, bufs, sems, blk, slot):
    pltpu.make_async_copy(
        pred_ref.at[:, pl.ds(blk * _W, _W)], bufs.at[slot], sems.at[slot]
    ).start()


def _loss_kernel(tgt_ref, pred_ref, out_ref, bufs, tail_buf,
                 m_ref, se_ref, rs_ref, g_ref, sems, tail_sem):
    m_ref[...] = jnp.full_like(m_ref, float("-inf"))
    se_ref[...] = jnp.zeros_like(se_ref)
    rs_ref[...] = jnp.zeros_like(rs_ref)
    g_ref[...] = jnp.zeros_like(g_ref)

    for b in range(_NBUF):
        _copy(pred_ref, bufs, sems, b, b)
    pltpu.make_async_copy(
        pred_ref.at[:, pl.ds(_NBT * _W, _TAIL)], tail_buf, tail_sem
    ).start()

    tgt = tgt_ref[...]

    def _update(x, col0):
        col = jax.lax.broadcasted_iota(jnp.int32, x.shape, 1)
        m = m_ref[...]
        bm = jnp.max(x, axis=1, keepdims=True)
        nm = jnp.maximum(m, bm)
        se_ref[...] = se_ref[...] * jnp.exp(m - nm) + jnp.sum(
            jnp.exp(x - nm), axis=1, keepdims=True
        )
        m_ref[...] = nm
        rs_ref[...] += jnp.sum(x, axis=1, keepdims=True)
        g_ref[...] += jnp.sum(
            jnp.where(col == tgt - col0, x, 0.0), axis=1, keepdims=True
        )

    def _step(blk, _):
        slot = lax.rem(blk, _NBUF)
        pltpu.make_async_copy(
            pred_ref.at[:, pl.ds(blk * _W, _W)], bufs.at[slot], sems.at[slot]
        ).wait()
        _update(bufs[slot], blk * _W)
        nxt = blk + _NBUF

        @pl.when(nxt < _NBT)
        def _start_next():
            _copy(pred_ref, bufs, sems, nxt, slot)

        return _

    lax.fori_loop(0, _NBT, _step, 0)

    pltpu.make_async_copy(
        pred_ref.at[:, pl.ds(_NBT * _W, _TAIL)], tail_buf, tail_sem
    ).wait()
    _update(tail_buf[...], _NBT * _W)

    z = m_ref[...] + jnp.log(se_ref[...])
    zsum = jnp.sum(z)
    kc = _SMOOTHING * math.log(_S) + _CONF * math.log(_CONF)
    total = (
        _B * kc
        - _S * (jnp.sum(rs_ref[...]) - _C * zsum)
        - (_CONF - _S) * (jnp.sum(g_ref[...]) - zsum)
    )
    out_ref[0, 0] = total / (_B * _C)


def kernel(pred, target):
    tgt = target.astype(jnp.int32).reshape(_B, 1)
    out = pl.pallas_call(
        _loss_kernel,
        in_specs=[
            pl.BlockSpec((_B, 1), lambda: (0, 0)),
            pl.BlockSpec(memory_space=pl.ANY),
        ],
        out_specs=pl.BlockSpec(
            (1, 1), lambda: (0, 0), memory_space=pltpu.SMEM
        ),
        out_shape=jax.ShapeDtypeStruct((1, 1), jnp.float32),
        scratch_shapes=[
            pltpu.VMEM((_NBUF, _B, _W), jnp.float32),
            pltpu.VMEM((_B, _TAIL), jnp.float32),
            pltpu.VMEM((_B, 1), jnp.float32),
            pltpu.VMEM((_B, 1), jnp.float32),
            pltpu.VMEM((_B, 1), jnp.float32),
            pltpu.VMEM((_B, 1), jnp.float32),
            pltpu.SemaphoreType.DMA((_NBUF,)),
            pltpu.SemaphoreType.DMA,
        ],
    )(tgt, pred)
    return out[0, 0]


# speedup vs baseline: 1.0015x; 1.0013x over previous
import math

import jax
import jax.numpy as jnp
from jax import lax
from jax.experimental import pallas as pl
from jax.experimental.pallas import tpu as pltpu

_C = 100000
_B = 1024
_SMOOTHING = 0.1
_CONF = 1.0 - _SMOOTHING
_S = _SMOOTHING / (_C - 1)
_W = 1024
_NBT = _C // _W
_TAIL = _C - _NBT * _W
_NBUF = 8


def _copy(pred_ref, bufs, sems, blk, slot):
    pltpu.make_async_copy(
        pred_ref.at[:, pl.ds(blk * _W, _W)], bufs.at[slot], sems.at[slot]
    ).start()


def _loss_kernel(tgt_ref, pred_ref, out_ref, bufs, tail_buf,
                 m_ref, se_ref, rs_ref, g_ref, sems, tail_sem):
    m_ref[...] = jnp.full_like(m_ref, float("-inf"))
    se_ref[...] = jnp.zeros_like(se_ref)
    rs_ref[...] = jnp.zeros_like(rs_ref)
    g_ref[...] = jnp.zeros_like(g_ref)

    for b in range(_NBUF):
        _copy(pred_ref, bufs, sems, b, b)
    pltpu.make_async_copy(
        pred_ref.at[:, pl.ds(_NBT * _W, _TAIL)], tail_buf, tail_sem
    ).start()

    tgt = tgt_ref[...]

    def _update(x, col0):
        col = jax.lax.broadcasted_iota(jnp.int32, x.shape, 1)
        m = m_ref[...]
        bm = jnp.max(x, axis=1, keepdims=True)
        nm = jnp.maximum(m, bm)
        se_ref[...] = se_ref[...] * jnp.exp(m - nm) + jnp.sum(
            jnp.exp(x - nm), axis=1, keepdims=True
        )
        m_ref[...] = nm
        rs_ref[...] += jnp.sum(x, axis=1, keepdims=True)
        g_ref[...] += jnp.sum(
            jnp.where(col == tgt - col0, x, 0.0), axis=1, keepdims=True
        )

    def _step(blk, _):
        slot = lax.rem(blk, _NBUF)
        pltpu.make_async_copy(
            pred_ref.at[:, pl.ds(blk * _W, _W)], bufs.at[slot], sems.at[slot]
        ).wait()
        _update(bufs[slot], blk * _W)
        nxt = blk + _NBUF

        @pl.when(nxt < _NBT)
        def _start_next():
            _copy(pred_ref, bufs, sems, nxt, slot)

        return _

    lax.fori_loop(0, _NBT, _step, 0)

    pltpu.make_async_copy(
        pred_ref.at[:, pl.ds(_NBT * _W, _TAIL)], tail_buf, tail_sem
    ).wait()
    _update(tail_buf[...], _NBT * _W)

    z = m_ref[...] + jnp.log(se_ref[...])
    zsum = jnp.sum(z)
    kc = _SMOOTHING * math.log(_S) + _CONF * math.log(_CONF)
    total = (
        _B * kc
        - _S * (jnp.sum(rs_ref[...]) - _C * zsum)
        - (_CONF - _S) * (jnp.sum(g_ref[...]) - zsum)
    )
    out_ref[0, 0] = total / (_B * _C)


def kernel(pred, target):
    tgt = target.astype(jnp.int32).reshape(_B, 1)
    out = pl.pallas_call(
        _loss_kernel,
        in_specs=[
            pl.BlockSpec((_B, 1), lambda: (0, 0)),
            pl.BlockSpec(memory_space=pl.ANY),
        ],
        out_specs=pl.BlockSpec(
            (1, 1), lambda: (0, 0), memory_space=pltpu.SMEM
        ),
        out_shape=jax.ShapeDtypeStruct((1, 1), jnp.float32),
        scratch_shapes=[
            pltpu.VMEM((_NBUF, _B, _W), jnp.float32),
            pltpu.VMEM((_B, _TAIL), jnp.float32),
            pltpu.VMEM((_B, 1), jnp.float32),
            pltpu.VMEM((_B, 1), jnp.float32),
            pltpu.VMEM((_B, 1), jnp.float32),
            pltpu.VMEM((_B, 1), jnp.float32),
            pltpu.SemaphoreType.DMA((_NBUF,)),
            pltpu.SemaphoreType.DMA,
        ],
    )(tgt, pred)
    return out[0, 0]
